# baseline (speedup 1.0000x reference)
import jax
import jax.numpy as jnp
from jax import lax
from jax.experimental import pallas as pl
from jax.experimental.pallas import tpu as pltpu
from jax.experimental.pallas import tpu_sc as plsc

_K = 9
_KN = _K - 1
_NEG_INF = float("-inf")
_SC_CORES = 2
_NPW = 32
_CHUNK = 16


def _tc1_body(xc_ref, w_ref, idx_ref, yut_ref, yv_ref):
    xc = xc_ref[...]
    n = xc.shape[1]
    cg = xc.shape[0] // 2

    nrm = jnp.sqrt(jnp.sum(xc * xc, axis=0, keepdims=True))
    xn = xc / jnp.maximum(nrm, 1e-12)
    x_sq = jnp.sum(xn * xn, axis=0, keepdims=True)
    inner = lax.dot_general(xn, xn, (((0,), (0,)), ((), ())),
                            preferred_element_type=jnp.float32)

    w0 = w_ref[0:cg]
    w1 = w_ref[cg:2 * cg]
    w2 = w_ref[2 * cg:3 * cg]
    w3 = w_ref[3 * cg:4 * cg]
    xa = xc[:cg]
    xb = xc[cg:]

    def mm_t(wb, xp):
        return lax.dot_general(wb, xp, (((1,), (0,)), ((), ())),
                               preferred_element_type=jnp.float32)

    def mm(xp, wb):
        return lax.dot_general(xp, wb, (((0,), (1,)), ((), ())),
                               preferred_element_type=jnp.float32)

    yut_ref[...] = jnp.concatenate([mm_t(w0, xa), mm_t(w1, xb)], axis=0)
    yv_ref[...] = jnp.concatenate([mm(xa, w2), mm(xb, w3)], axis=1)

    row = lax.broadcasted_iota(jnp.int32, (n, n), 0)
    col = lax.broadcasted_iota(jnp.int32, (n, n), 1)
    nd = 2.0 * inner - jnp.transpose(x_sq)
    nd = jnp.where(row == col, _NEG_INF, nd)
    bi0 = lax.broadcasted_iota(jnp.int32, (128, n), 0)

    def body(k, nd):
        bv = nd[0:128]
        bi = bi0
        for j in range(1, n // 128):
            v = nd[j * 128:(j + 1) * 128]
            c = v > bv
            bv = jnp.where(c, v, bv)
            bi = jnp.where(c, bi0 + j * 128, bi)
        m = jnp.max(bv, axis=0, keepdims=True)
        sel = jnp.min(jnp.where(bv == m, bi, n), axis=0, keepdims=True)
        idx_ref[pl.ds(k, 1), :] = sel
        return jnp.where(row == sel, _NEG_INF, nd)

    lax.fori_loop(0, _KN, body, nd)


def _sc_body(yv_hbm, idx_hbm, agg_hbm,
             idx_v, idx_v2, rows_a, rows_b, self_a, self_b, out_v,
             sem_a, sem_b):
    wid = lax.axis_index("s") * _SC_CORES + lax.axis_index("c")
    nbase = wid * _NPW
    nchunks = _NPW // _CHUNK
    idxs = [idx_v, idx_v2]
    for c in range(nchunks):
        pltpu.sync_copy(idx_hbm.at[wid, c], idxs[c])

    bufs = [(rows_a, self_a, sem_a), (rows_b, self_b, sem_b)]

    def fire(c):
        buf, selfb, sem = bufs[c % 2]
        h1 = pltpu.async_copy(yv_hbm.at[idxs[c]], buf, sem)
        h2 = pltpu.async_copy(
            yv_hbm.at[pl.ds(nbase + c * _CHUNK, _CHUNK)], selfb, sem)
        return (h1, h2)

    pending = fire(0)
    for c in range(nchunks):
        nxt = fire(c + 1) if c + 1 < nchunks else None
        for h in pending:
            h.wait()
        pending = nxt
        buf, selfb, _ = bufs[c % 2]

        @pl.loop(0, _CHUNK)
        def _node_loop(i):
            for c0 in range(0, 384, 16):
                acc = selfb.at[i, pl.ds(c0, 16)][...]
                for k in range(_KN):
                    acc = jnp.maximum(
                        acc, buf.at[k * _CHUNK + i, pl.ds(c0, 16)][...])
                out_v.at[i, pl.ds(c0, 16)][...] = acc

        pltpu.sync_copy(out_v,
                        agg_hbm.at[pl.ds(nbase + c * _CHUNK, _CHUNK)])


def _tc2_body(agg_ref, yv_ref, yut_ref, b_ref, out_ref):
    half = yut_ref.shape[0]
    d = agg_ref[...] - yv_ref[...]
    dt = jnp.transpose(d)
    out_ref[0:half, :] = jnp.maximum(yut_ref[...] + b_ref[0:half], 0.0)
    out_ref[half:, :] = jnp.maximum(dt + b_ref[half:], 0.0)


def _sc_gather_max(yv_b, idx_b):
    n, ch = yv_b.shape
    nchunks = _NPW // _CHUNK
    idx_sc = (idx_b.reshape(_KN, 32, nchunks, _CHUNK)
              .transpose(1, 2, 0, 3).reshape(32, nchunks, _KN * _CHUNK))
    f = pl.kernel(
        _sc_body,
        out_type=jax.ShapeDtypeStruct((n, ch), jnp.float32),
        mesh=plsc.VectorSubcoreMesh(core_axis_name="c",
                                    subcore_axis_name="s"),
        scratch_types=[
            pltpu.VMEM((_KN * _CHUNK,), jnp.int32),
            pltpu.VMEM((_KN * _CHUNK,), jnp.int32),
            pltpu.VMEM((_KN * _CHUNK, 384), jnp.float32),
            pltpu.VMEM((_KN * _CHUNK, 384), jnp.float32),
            pltpu.VMEM((_CHUNK, 384), jnp.float32),
            pltpu.VMEM((_CHUNK, 384), jnp.float32),
            pltpu.VMEM((_CHUNK, 384), jnp.float32),
            pltpu.SemaphoreType.DMA,
            pltpu.SemaphoreType.DMA,
        ],
    )
    return f(yv_b, idx_sc)


@jax.jit
def kernel(x, conv_w, conv_b):
    B, C, H, W = x.shape
    N = H * W
    Cout = conv_w.shape[0]
    half = Cout // 2
    xf = x.reshape(B, C, N)
    bias_col = conv_b.reshape(Cout, 1)

    tc1 = pl.pallas_call(
        _tc1_body,
        out_shape=[
            jax.ShapeDtypeStruct((_KN, N), jnp.int32),
            jax.ShapeDtypeStruct((half, N), jnp.float32),
            jax.ShapeDtypeStruct((N, half), jnp.float32),
        ],
    )

    tc2 = pl.pallas_call(
        _tc2_body,
        out_shape=jax.ShapeDtypeStruct((Cout, N), jnp.float32),
    )

    outs = []
    for b in range(B):
        idx_b, yut_b, yv_b = tc1(xf[b], conv_w)
        agg_b = _sc_gather_max(yv_b, idx_b)
        outs.append(tc2(agg_b, yv_b, yut_b, bias_col))

    return jnp.stack(outs).reshape(B, Cout, H, W)


# speedup vs baseline: 1.0327x; 1.0327x over previous
import jax
import jax.numpy as jnp
from jax import lax
from jax.experimental import pallas as pl
from jax.experimental.pallas import tpu as pltpu
from jax.experimental.pallas import tpu_sc as plsc

_K = 9
_KN = _K - 1
_NEG_INF = float("-inf")
_SC_CORES = 2
_NPW = 32
_CHUNK = 16


def _tc1_body(xc_ref, w_ref, idx_ref, yut_ref, yv_ref):
    xc = xc_ref[...]
    n = xc.shape[1]
    cg = xc.shape[0] // 2

    nrm = jnp.sqrt(jnp.sum(xc * xc, axis=0, keepdims=True))
    xn = xc / jnp.maximum(nrm, 1e-12)
    x_sq = jnp.sum(xn * xn, axis=0, keepdims=True)
    inner = lax.dot_general(xn, xn, (((0,), (0,)), ((), ())),
                            preferred_element_type=jnp.float32)

    w0 = w_ref[0:cg]
    w1 = w_ref[cg:2 * cg]
    w2 = w_ref[2 * cg:3 * cg]
    w3 = w_ref[3 * cg:4 * cg]
    xa = xc[:cg]
    xb = xc[cg:]

    def mm_t(wb, xp):
        return lax.dot_general(wb, xp, (((1,), (0,)), ((), ())),
                               preferred_element_type=jnp.float32)

    def mm(xp, wb):
        return lax.dot_general(xp, wb, (((0,), (1,)), ((), ())),
                               preferred_element_type=jnp.float32)

    yut_ref[...] = jnp.concatenate([mm_t(w0, xa), mm_t(w1, xb)], axis=0)
    yv_ref[...] = jnp.concatenate([mm(xa, w2), mm(xb, w3)], axis=1)

    row = lax.broadcasted_iota(jnp.int32, (n, n), 0)
    col = lax.broadcasted_iota(jnp.int32, (n, n), 1)
    nd = 2.0 * inner - jnp.transpose(x_sq)
    nd = jnp.where(row == col, _NEG_INF, nd)
    def body(k, nd):
        m = jnp.max(nd, axis=0, keepdims=True)
        sel = jnp.min(jnp.where(nd == m, row, n), axis=0, keepdims=True)
        idx_ref[pl.ds(k, 1), :] = sel
        return jnp.where(row == sel, _NEG_INF, nd)

    lax.fori_loop(0, _KN, body, nd)


def _sc_body(yv_hbm, idx_hbm, agg_hbm,
             idx_v, idx_v2, rows_a, rows_b, self_a, self_b, out_v,
             sem_a, sem_b):
    wid = lax.axis_index("s") * _SC_CORES + lax.axis_index("c")
    nbase = wid * _NPW
    nchunks = _NPW // _CHUNK
    idxs = [idx_v, idx_v2]
    for c in range(nchunks):
        pltpu.sync_copy(idx_hbm.at[wid, c], idxs[c])

    bufs = [(rows_a, self_a, sem_a), (rows_b, self_b, sem_b)]

    def fire(c):
        buf, selfb, sem = bufs[c % 2]
        h1 = pltpu.async_copy(yv_hbm.at[idxs[c]], buf, sem)
        h2 = pltpu.async_copy(
            yv_hbm.at[pl.ds(nbase + c * _CHUNK, _CHUNK)], selfb, sem)
        return (h1, h2)

    pending = fire(0)
    for c in range(nchunks):
        nxt = fire(c + 1) if c + 1 < nchunks else None
        for h in pending:
            h.wait()
        pending = nxt
        buf, selfb, _ = bufs[c % 2]

        @pl.loop(0, _CHUNK)
        def _node_loop(i):
            for c0 in range(0, 384, 16):
                acc = selfb.at[i, pl.ds(c0, 16)][...]
                for k in range(_KN):
                    acc = jnp.maximum(
                        acc, buf.at[k * _CHUNK + i, pl.ds(c0, 16)][...])
                out_v.at[i, pl.ds(c0, 16)][...] = acc

        pltpu.sync_copy(out_v,
                        agg_hbm.at[pl.ds(nbase + c * _CHUNK, _CHUNK)])


def _tc2_body(agg_ref, yv_ref, yut_ref, b_ref, out_ref):
    half = yut_ref.shape[0]
    d = agg_ref[...] - yv_ref[...]
    dt = jnp.transpose(d)
    out_ref[0:half, :] = jnp.maximum(yut_ref[...] + b_ref[0:half], 0.0)
    out_ref[half:, :] = jnp.maximum(dt + b_ref[half:], 0.0)


def _sc_gather_max(yv_b, idx_b):
    n, ch = yv_b.shape
    nchunks = _NPW // _CHUNK
    idx_sc = (idx_b.reshape(_KN, 32, nchunks, _CHUNK)
              .transpose(1, 2, 0, 3).reshape(32, nchunks, _KN * _CHUNK))
    f = pl.kernel(
        _sc_body,
        out_type=jax.ShapeDtypeStruct((n, ch), jnp.float32),
        mesh=plsc.VectorSubcoreMesh(core_axis_name="c",
                                    subcore_axis_name="s"),
        scratch_types=[
            pltpu.VMEM((_KN * _CHUNK,), jnp.int32),
            pltpu.VMEM((_KN * _CHUNK,), jnp.int32),
            pltpu.VMEM((_KN * _CHUNK, 384), jnp.float32),
            pltpu.VMEM((_KN * _CHUNK, 384), jnp.float32),
            pltpu.VMEM((_CHUNK, 384), jnp.float32),
            pltpu.VMEM((_CHUNK, 384), jnp.float32),
            pltpu.VMEM((_CHUNK, 384), jnp.float32),
            pltpu.SemaphoreType.DMA,
            pltpu.SemaphoreType.DMA,
        ],
    )
    return f(yv_b, idx_sc)


@jax.jit
def kernel(x, conv_w, conv_b):
    B, C, H, W = x.shape
    N = H * W
    Cout = conv_w.shape[0]
    half = Cout // 2
    xf = x.reshape(B, C, N)
    bias_col = conv_b.reshape(Cout, 1)

    tc1 = pl.pallas_call(
        _tc1_body,
        out_shape=[
            jax.ShapeDtypeStruct((_KN, N), jnp.int32),
            jax.ShapeDtypeStruct((half, N), jnp.float32),
            jax.ShapeDtypeStruct((N, half), jnp.float32),
        ],
    )

    tc2 = pl.pallas_call(
        _tc2_body,
        out_shape=jax.ShapeDtypeStruct((Cout, N), jnp.float32),
    )

    outs = []
    for b in range(B):
        idx_b, yut_b, yv_b = tc1(xf[b], conv_w)
        agg_b = _sc_gather_max(yv_b, idx_b)
        outs.append(tc2(agg_b, yv_b, yut_b, bias_col))

    return jnp.stack(outs).reshape(B, Cout, H, W)
